# Initial kernel scaffold; baseline (speedup 1.0000x reference)
import jax
import jax.numpy as jnp
from jax.experimental import pallas as pl


def kernel(node_feat, node_opcode, edge_index, node_config_feat, node_config_ids, config_edge_index, params):
    raise NotImplementedError("write your pallas kernel here")


# speedup vs baseline: 1.5272x; 1.5272x over previous
import functools
import jax
import jax.numpy as jnp
import numpy as np
from jax.experimental import pallas as pl
from jax.experimental.pallas import tpu as pltpu


_HI = jax.lax.Precision.HIGHEST


def _dot(a, b):
    return jnp.dot(a, b, preferred_element_type=jnp.float32)


def _adot(a, b):
    return jnp.dot(a, b, precision=_HI, preferred_element_type=jnp.float32)


def _split(x):
    hi = x.astype(jnp.bfloat16)
    lo = (x - hi.astype(jnp.float32)).astype(jnp.bfloat16)
    return hi, lo


def _dot3(a, b):
    ah, al = _split(a)
    bh, bl = _split(b)
    d = lambda u, v: jnp.dot(u, v, preferred_element_type=jnp.float32)
    return d(ah, bh) + d(ah, bl) + d(al, bh)


def _dot3_pre(ah, al, b):
    bh, bl = _split(b)
    d = lambda u, v: jnp.dot(u, v, preferred_element_type=jnp.float32)
    return d(ah, bh) + d(ah, bl) + d(al, bh)


def _lrelu(x):
    return jnp.where(x >= 0, x, 0.01 * x)


def _bd(w, n):
    return jnp.kron(jnp.eye(n, dtype=w.dtype), w)


def _cfg_body(pf_ref, ahi_ref, alo_ref, c1_ref, pw_ref, pb_ref, wln_ref, wrn_ref,
              wl2_ref, wr2_ref, b2_ref, wl3_ref, wr3_ref, b3_ref,
              wl4_ref, wr4_ref, b4_ref, d1_ref, d2_ref, d3_ref, out_ref):
    cb = pf_ref.shape[0]
    ahi = ahi_ref[...]
    alo = alo_ref[...]
    pw = pw_ref[...]
    pb = pb_ref[...]
    ncf = jnp.concatenate(
        [_lrelu(_dot(pf_ref[c], pw) + pb)
         for c in range(cb)], axis=1)
    aggn = _dot3_pre(ahi, alo, ncf)
    h = _lrelu(c1_ref[...]
               + _dot(aggn, wln_ref[...])
               + _dot(ncf, wrn_ref[...]))
    for wl_r, wr_r, b_r in ((wl2_ref, wr2_ref, b2_ref),
                            (wl3_ref, wr3_ref, b3_ref),
                            (wl4_ref, wr4_ref, b4_ref)):
        m = _dot3_pre(ahi, alo, h)
        h = _lrelu(_dot(m, wl_r[...])
                   + b_r[...]
                   + _dot(h, wr_r[...]))
    pooled = jnp.mean(h, axis=0, keepdims=True)
    z = _lrelu(_dot(pooled, d1_ref[...]))
    z = _lrelu(_dot(z, d2_ref[...]))
    s = _dot(z, d3_ref[...])
    out_ref[0, 0, :] = s[0]


def _run_cfg_stack(node_config_feat, A, c1, params, cb):
    C, NCN, FP = node_config_feat.shape
    p1, p2, p3, p4 = params["cfg"]
    wln = _bd(p1["Wl"][128:192], cb)
    wrn = _bd(p1["Wr"][128:192], cb)
    a_hi, a_lo = _split(A)
    args = [
        node_config_feat,
        a_hi,
        a_lo,
        jnp.tile(c1, (1, cb)),
        params["prj_W"],
        params["prj_b"][None, :],
        wln, wrn,
    ]
    specs = [
        pl.BlockSpec((cb, NCN, FP), lambda i: (i, 0, 0)),
        pl.BlockSpec((NCN, NCN), lambda i: (0, 0)),
        pl.BlockSpec((NCN, NCN), lambda i: (0, 0)),
        pl.BlockSpec((NCN, cb * 64), lambda i: (0, 0)),
        pl.BlockSpec((FP, 64), lambda i: (0, 0)),
        pl.BlockSpec((1, 64), lambda i: (0, 0)),
        pl.BlockSpec((cb * 64, cb * 64), lambda i: (0, 0)),
        pl.BlockSpec((cb * 64, cb * 64), lambda i: (0, 0)),
    ]
    for p in (p2, p3, p4):
        args += [_bd(p["Wl"], cb), _bd(p["Wr"], cb),
                 jnp.tile(p["bl"], cb)[None, :]]
        specs += [pl.BlockSpec((cb * 64, cb * 64), lambda i: (0, 0)),
                  pl.BlockSpec((cb * 64, cb * 64), lambda i: (0, 0)),
                  pl.BlockSpec((1, cb * 64), lambda i: (0, 0))]
    args += [_bd(params["d1"], cb), _bd(params["d2"], cb),
             _bd(params["d3"], cb)]
    specs += [pl.BlockSpec((cb * 64, cb * 64), lambda i: (0, 0)),
              pl.BlockSpec((cb * 64, cb * 64), lambda i: (0, 0)),
              pl.BlockSpec((cb * 64, cb), lambda i: (0, 0))]
    out = pl.pallas_call(
        _cfg_body,
        grid=(C // cb,),
        in_specs=specs,
        out_specs=pl.BlockSpec((1, 1, cb), lambda i: (i, 0, 0)),
        out_shape=jax.ShapeDtypeStruct((C // cb, 1, cb), jnp.float32),
    )(*args)
    return out.reshape(C)


def _big_graph(node_feat, node_opcode, edge_index, params):
    N = node_feat.shape[0]
    src, dst = edge_index[0], edge_index[1]
    code = params["emb"][node_opcode]
    x = jnp.concatenate([node_feat, code], axis=1)
    x = x / jnp.maximum(jnp.linalg.norm(x, axis=-1, keepdims=True), 1e-12)
    cnt = jax.ops.segment_sum(
        jnp.ones((src.shape[0],), jnp.float32), dst, num_segments=N)
    den = jnp.maximum(cnt, 1.0)[:, None]
    for p in params["node"]:
        agg = jax.ops.segment_sum(x[src], dst, num_segments=N) / den
        x = _lrelu(_dot(agg, p["Wl"]) + p["bl"] + _dot(x, p["Wr"]))
    nbrs = jax.ops.segment_sum(x[src], dst, num_segments=N) / den
    return x, nbrs


def kernel(node_feat, node_opcode, edge_index, node_config_feat,
           node_config_ids, config_edge_index, params):
    C, NCN, _ = node_config_feat.shape
    cb = 4 if C % 4 == 0 else 1

    x, nbrs = _big_graph(node_feat, node_opcode, edge_index, params)

    csrc, cdst = config_edge_index[0], config_edge_index[1]
    acnt = jnp.zeros((NCN, NCN), jnp.float32).at[cdst, csrc].add(1.0)
    ccnt = jnp.sum(acnt, axis=1)
    A = acnt / jnp.maximum(ccnt, 1.0)[:, None]

    cn = nbrs[node_config_ids]
    xs = x[node_config_ids]
    for p in params["nbr"]:
        cn = _lrelu(_dot(_adot(A, cn), p["Wl"]) + p["bl"] + _dot(cn, p["Wr"]))

    p1 = params["cfg"][0]
    wl1, wr1 = p1["Wl"], p1["Wr"]
    c1 = (_dot(_adot(A, cn), wl1[0:64]) + _dot(_adot(A, xs), wl1[64:128])
          + _dot(cn, wr1[0:64]) + _dot(xs, wr1[64:128]) + p1["bl"])

    return _run_cfg_stack(node_config_feat, A, c1, params, cb)
